# baseline (speedup 1.0000x reference)
import functools

import jax
import jax.numpy as jnp
from jax import lax
from jax.experimental import pallas as pl
from jax.experimental.pallas import tpu as pltpu
from jax.experimental.pallas import tpu_sc as plsc

_VOCAB = 100000
_DIM = 300
_BATCH = 4096
_MAXLEN = 50
_LPAD = 56
_DPAD = 384
_NZERO = 4096
_VEXT = 106000
_PADBLK = 2000
_SLCBLK = 32


def _pad_table(table):
    def body(t_ref, o_ref):
        i = pl.program_id(0)
        row = i * _PADBLK + lax.broadcasted_iota(jnp.int32, (_PADBLK, _DPAD), 0)
        data = jnp.concatenate(
            [t_ref[...], jnp.zeros((_PADBLK, _DPAD - _DIM), jnp.float32)], axis=1
        )
        o_ref[...] = jnp.where(row < _VOCAB, data, 0.0)

    nin = _VOCAB // _PADBLK
    return pl.pallas_call(
        body,
        grid=(_VEXT // _PADBLK,),
        in_specs=[
            pl.BlockSpec(
                (_PADBLK, _DIM), lambda i: (jnp.minimum(i, nin - 1), 0)
            )
        ],
        out_specs=pl.BlockSpec((_PADBLK, _DPAD), lambda i: (i, 0)),
        out_shape=jax.ShapeDtypeStruct((_VEXT, _DPAD), jnp.float32),
    )(table)


def _mask_and_midx(idx32, len2d):
    def body(idx_ref, len_ref, mask_ref, midx_ref):
        pos = lax.broadcasted_iota(jnp.int32, (_BATCH, _MAXLEN), 1)
        valid = pos < len_ref[...]
        mask_ref[...] = valid.astype(jnp.float32)

        pos56 = lax.broadcasted_iota(jnp.int32, (_BATCH, _LPAD), 1)
        row56 = lax.broadcasted_iota(jnp.int32, (_BATCH, _LPAD), 0)
        valid56 = pos56 < len_ref[...]
        idx56 = jnp.concatenate(
            [idx_ref[...], jnp.zeros((_BATCH, _LPAD - _MAXLEN), jnp.int32)],
            axis=1,
        )
        zrow = _VOCAB + ((row56 * _LPAD + pos56) & (_NZERO - 1))
        midx_ref[...] = jnp.where(valid56, idx56, zrow)

    return pl.pallas_call(
        body,
        out_shape=(
            jax.ShapeDtypeStruct((_BATCH, _MAXLEN), jnp.float32),
            jax.ShapeDtypeStruct((_BATCH, _LPAD), jnp.int32),
        ),
    )(idx32, len2d)


def _sc_gather(table_ext, midx):
    mesh = plsc.VectorSubcoreMesh(core_axis_name="c", subcore_axis_name="s")

    @functools.partial(
        pl.kernel,
        out_type=jax.ShapeDtypeStruct((_BATCH, _LPAD, _DPAD), jnp.float32),
        mesh=mesh,
    )
    def k(table_hbm, idx_hbm, out_hbm):
        def body(i_vmem, o_vmem):
            pltpu.sync_copy(table_hbm.at[i_vmem.at[0]], o_vmem.at[0])

        pltpu.emit_pipeline(
            body,
            grid=(_BATCH,),
            in_specs=[pl.BlockSpec((1, _LPAD), index_map=lambda i: (i, 0))],
            out_specs=[
                pl.BlockSpec((1, _LPAD, _DPAD), index_map=lambda i: (i, 0, 0))
            ],
            core_axis_name=("c", "s"),
            dimension_semantics=(pltpu.PARALLEL,),
        )(idx_hbm, out_hbm)

    return k(table_ext, midx)


def _slice_out(emb_pad):
    def body(i_ref, o_ref):
        o_ref[...] = i_ref[:, : _MAXLEN, : _DIM]

    return pl.pallas_call(
        body,
        grid=(_BATCH // _SLCBLK,),
        in_specs=[pl.BlockSpec((_SLCBLK, _LPAD, _DPAD), lambda i: (i, 0, 0))],
        out_specs=pl.BlockSpec((_SLCBLK, _MAXLEN, _DIM), lambda i: (i, 0, 0)),
        out_shape=jax.ShapeDtypeStruct((_BATCH, _MAXLEN, _DIM), jnp.float32),
    )(emb_pad)


def kernel(table, indices, lengths):
    idx32 = indices.astype(jnp.int32)
    table_ext = _pad_table(table)
    mask, midx = _mask_and_midx(idx32, lengths.reshape(_BATCH, 1))
    emb_pad = _sc_gather(table_ext, midx)
    emb = _slice_out(emb_pad)
    return emb, mask


# speedup vs baseline: 1.4127x; 1.4127x over previous
import functools

import jax
import jax.numpy as jnp
from jax import lax
from jax.experimental import pallas as pl
from jax.experimental.pallas import tpu as pltpu
from jax.experimental.pallas import tpu_sc as plsc

_VOCAB = 100000
_DIM = 300
_BATCH = 4096
_MAXLEN = 50
_LPAD = 56
_DPAD = 384
_NZERO = 4096
_VEXT = 106000
_PADBLK = 2000
_SLCBLK = 32


def _pad_table(table):
    def body(t_ref, o_ref):
        i = pl.program_id(0)
        row = i * _PADBLK + lax.broadcasted_iota(jnp.int32, (_PADBLK, _DPAD), 0)
        data = jnp.concatenate(
            [t_ref[...], jnp.zeros((_PADBLK, _DPAD - _DIM), jnp.float32)], axis=1
        )
        o_ref[...] = jnp.where(row < _VOCAB, data, 0.0)

    nin = _VOCAB // _PADBLK
    return pl.pallas_call(
        body,
        grid=(_VEXT // _PADBLK,),
        in_specs=[
            pl.BlockSpec(
                (_PADBLK, _DIM), lambda i: (jnp.minimum(i, nin - 1), 0)
            )
        ],
        out_specs=pl.BlockSpec((_PADBLK, _DPAD), lambda i: (i, 0)),
        out_shape=jax.ShapeDtypeStruct((_VEXT, _DPAD), jnp.float32),
    )(table)


def _mask_and_midx(idx32, len2d):
    def body(idx_ref, len_ref, mask_ref, midx_ref):
        pos = lax.broadcasted_iota(jnp.int32, (_BATCH, _MAXLEN), 1)
        valid = pos < len_ref[...]
        mask_ref[...] = valid.astype(jnp.float32)

        pos56 = lax.broadcasted_iota(jnp.int32, (_BATCH, _LPAD), 1)
        row56 = lax.broadcasted_iota(jnp.int32, (_BATCH, _LPAD), 0)
        valid56 = pos56 < len_ref[...]
        idx56 = jnp.concatenate(
            [idx_ref[...], jnp.zeros((_BATCH, _LPAD - _MAXLEN), jnp.int32)],
            axis=1,
        )
        zrow = _VOCAB + ((row56 * _LPAD + pos56) & (_NZERO - 1))
        midx_ref[...] = jnp.where(valid56, idx56, zrow)

    return pl.pallas_call(
        body,
        out_shape=(
            jax.ShapeDtypeStruct((_BATCH, _MAXLEN), jnp.float32),
            jax.ShapeDtypeStruct((_BATCH, _LPAD), jnp.int32),
        ),
    )(idx32, len2d)


def _sc_gather(table_ext, midx):
    mesh = plsc.VectorSubcoreMesh(core_axis_name="c", subcore_axis_name="s")

    @functools.partial(
        pl.kernel,
        out_type=jax.ShapeDtypeStruct((_BATCH, _LPAD, _DPAD), jnp.float32),
        mesh=mesh,
    )
    def k(table_hbm, idx_hbm, out_hbm):
        def body(i_vmem, o_vmem):
            pltpu.sync_copy(table_hbm.at[i_vmem.at[0]], o_vmem.at[0])

        pltpu.emit_pipeline(
            body,
            grid=(_BATCH,),
            in_specs=[pl.BlockSpec((1, _LPAD), index_map=lambda i: (i, 0))],
            out_specs=[
                pl.BlockSpec((1, _LPAD, _DPAD), index_map=lambda i: (i, 0, 0))
            ],
            core_axis_name=("c", "s"),
            dimension_semantics=(pltpu.PARALLEL,),
        )(idx_hbm, out_hbm)

    return k(table_ext, midx)


def _slice_out(emb_pad):
    def body(i_ref, o_ref):
        o_ref[...] = i_ref[:, : _MAXLEN, : _DIM]

    return pl.pallas_call(
        body,
        grid=(_BATCH // _SLCBLK,),
        in_specs=[pl.BlockSpec((_SLCBLK, _LPAD, _DPAD), lambda i: (i, 0, 0))],
        out_specs=pl.BlockSpec((_SLCBLK, _MAXLEN, _DIM), lambda i: (i, 0, 0)),
        out_shape=jax.ShapeDtypeStruct((_BATCH, _MAXLEN, _DIM), jnp.float32),
    )(emb_pad)


def kernel(table, indices, lengths):
    idx32 = indices.astype(jnp.int32)
    table_ext = _pad_table(table)
    mask, midx = _mask_and_midx(idx32, lengths.reshape(_BATCH, 1))
    emb_pad = _sc_gather(table_ext, midx)
    emb = emb_pad[:, :_MAXLEN, :_DIM]
    return emb, mask
